# baseline (speedup 1.0000x reference)
import jax
import jax.numpy as jnp
from jax import lax
from jax.experimental import pallas as pl
from jax.experimental.pallas import tpu as pltpu
from jax.experimental.pallas import tpu_sc as plsc

BATCH = 4096
CON = 13
CAT = 26
VOCAB = 100000
DIM = 64
NQ = DIM // 16

NUM_CORES = 2
NUM_SUBCORES = 16
NW = NUM_CORES * NUM_SUBCORES
BPW = BATCH // NW
RPW = BPW * CAT
SUB = 16
NSUB = BPW // SUB
SROWS = SUB * CAT
GCH = 104
NG = SROWS // GCH
DDIM = (1 + CON) * DIM


def _body(xcon_hbm, idx_hbm, cls_hbm, conW_hbm, conb_hbm, pairs_hbm,
          dense_hbm, cat_hbm, idx_v, par_v, xcon_v, cls_v, conW_v, conb_v,
          catpad, catbuf, densebuf, gsem):
    w = lax.axis_index("s") * NUM_CORES + lax.axis_index("c")
    b0 = pl.multiple_of(w * BPW, BPW)

    pltpu.sync_copy(idx_hbm.at[pl.ds(b0 * CAT, RPW)], idx_v)
    pltpu.sync_copy(xcon_hbm.at[pl.ds(b0 * CON, BPW * CON)], xcon_v)
    pltpu.sync_copy(cls_hbm, cls_v)
    pltpu.sync_copy(conW_hbm, conW_v)
    pltpu.sync_copy(conb_hbm, conb_v)

    def split_body(i, _):
        v = idx_v[pl.ds(i * 16, 16)]
        par_v[pl.ds(i * 16, 16)] = lax.bitwise_and(v, 1)
        idx_v[pl.ds(i * 16, 16)] = lax.shift_right_logical(v, 1)
        return _

    lax.fori_loop(0, RPW // 16, split_body, None)

    cls_regs = [cls_v[pl.ds(q * 16, 16)] for q in range(NQ)]

    for c in range(NSUB):
        bb = c * SUB
        r0 = c * SROWS
        handles = [
            pltpu.async_copy(
                pairs_hbm.at[idx_v.at[pl.ds(r0 + g * GCH, GCH)]],
                catpad.at[pl.ds(g * GCH, GCH)], gsem)
            for g in range(NG)
        ]

        def cls_body(b, _):
            for q in range(NQ):
                densebuf[pl.ds(b * DDIM + q * 16, 16)] = cls_regs[q]
            return _

        lax.fori_loop(0, SUB, cls_body, None)

        for f in range(CON):
            w_regs = [conW_v[pl.ds(f * DIM + q * 16, 16)] for q in range(NQ)]
            b_regs = [conb_v[pl.ds(f * DIM + q * 16, 16)] for q in range(NQ)]

            def con_body(b, _, f=f, w_regs=w_regs, b_regs=b_regs):
                xb = plsc.load_gather(
                    xcon_v,
                    [jnp.full((16,), (bb + b) * CON + f, dtype=jnp.int32)])
                for q in range(NQ):
                    densebuf[pl.ds(b * DDIM + (1 + f) * DIM + q * 16,
                                   16)] = jnp.maximum(
                        xb * w_regs[q] + b_regs[q], 0.0)
                return _

            lax.fori_loop(0, SUB, con_body, None)

        for h in handles:
            h.wait()

        def ext_body(i, _):
            parb = plsc.load_gather(
                par_v, [jnp.full((16,), r0 + i, dtype=jnp.int32)])
            m = parb != 0
            for q in range(NQ):
                lo = catpad[i, pl.ds(q * 16, 16)]
                hi = catpad[i, pl.ds(DIM + q * 16, 16)]
                catbuf[pl.ds(i * DIM + q * 16, 16)] = jnp.where(m, hi, lo)
            return _

        lax.fori_loop(0, SROWS, ext_body, None)

        pltpu.sync_copy(densebuf,
                        dense_hbm.at[pl.ds((b0 + bb) * DDIM, SUB * DDIM)])
        pltpu.sync_copy(catbuf,
                        cat_hbm.at[pl.ds((b0 * CAT + r0) * DIM, SROWS * DIM)])


@jax.jit
def _sc_call(xcon, flat_idx, cls_flat, conW, conb, pairs):
    mesh = plsc.VectorSubcoreMesh(core_axis_name="c", subcore_axis_name="s")
    kern = pl.kernel(
        _body,
        out_type=(
            jax.ShapeDtypeStruct((BATCH * DDIM,), jnp.float32),
            jax.ShapeDtypeStruct((BATCH * CAT * DIM,), jnp.float32),
        ),
        mesh=mesh,
        compiler_params=pltpu.CompilerParams(needs_layout_passes=False),
        scratch_types=[
            pltpu.VMEM((RPW,), jnp.int32),
            pltpu.VMEM((RPW,), jnp.int32),
            pltpu.VMEM((BPW * CON,), jnp.float32),
            pltpu.VMEM((DIM,), jnp.float32),
            pltpu.VMEM((CON * DIM,), jnp.float32),
            pltpu.VMEM((CON * DIM,), jnp.float32),
            pltpu.VMEM((SROWS, 2 * DIM), jnp.float32),
            pltpu.VMEM((SROWS * DIM,), jnp.float32),
            pltpu.VMEM((SUB * DDIM,), jnp.float32),
            pltpu.SemaphoreType.DMA,
        ],
    )
    return kern(xcon, flat_idx, cls_flat, conW, conb, pairs)


def kernel(x_con, x_cat, cls, con_W, con_b, tables):
    offs = (jnp.arange(CAT, dtype=jnp.int32) * VOCAB)[None, :]
    flat_idx = (x_cat.astype(jnp.int32) + offs).reshape(BATCH * CAT)
    pairs = tables.reshape(CAT * VOCAB // 2, 2 * DIM)
    dense, cat = _sc_call(x_con.reshape(BATCH * CON), flat_idx,
                          cls.reshape(DIM), con_W.reshape(CON * DIM),
                          con_b.reshape(CON * DIM), pairs)
    return jnp.concatenate(
        [dense.reshape(BATCH, 1 + CON, DIM),
         cat.reshape(BATCH, CAT, DIM)], axis=1)


# speedup vs baseline: 4.5648x; 4.5648x over previous
import jax
import jax.numpy as jnp
from jax import lax
from jax.experimental import pallas as pl
from jax.experimental.pallas import tpu as pltpu
from jax.experimental.pallas import tpu_sc as plsc

BATCH = 4096
CON = 13
CAT = 26
VOCAB = 100000
DIM = 64
ROWS = 1 + CON + CAT

NUM_CORES = 2
NUM_SUBCORES = 16
NW = NUM_CORES * NUM_SUBCORES
NCH = BATCH // 16


def _body(xconF_hbm, xcatT_hbm, cls_hbm, conW_hbm, conb_hbm, t_hbm, out_hbm,
          idx_a, idx_b, xcon_v, cls_v, conW_v, conb_v, slab_v, row_a, row_b,
          ssem, wsem, isem):
    wid = lax.axis_index("s") * NUM_CORES + lax.axis_index("c")

    pltpu.sync_copy(cls_hbm, cls_v)
    pltpu.sync_copy(conW_hbm, conW_v)
    pltpu.sync_copy(conb_hbm, conb_v)

    rowbufs = [row_a, row_b]
    pend = [None, None]
    state = {"t": 0}

    def emit_row(r, d, fill):
        i = state["t"]
        state["t"] ^= 1
        if pend[i] is not None:
            pend[i].wait()
        fill(rowbufs[i])
        pend[i] = pltpu.async_copy(rowbufs[i], out_hbm.at[r, d, :], wsem)

    h_slab = pltpu.async_copy(t_hbm.at[0, wid, :], slab_v, ssem)

    for f in range(CON):
        pltpu.sync_copy(xconF_hbm.at[pl.ds(f * BATCH, BATCH)], xcon_v)
        for dj in range(2):
            d = wid + 32 * dj
            wv = plsc.load_gather(
                conW_v, [jnp.full((16,), f * DIM + d, dtype=jnp.int32)])
            bv = plsc.load_gather(
                conb_v, [jnp.full((16,), f * DIM + d, dtype=jnp.int32)])

            def fill(buf, wv=wv, bv=bv):
                def bdy(i, _):
                    buf[pl.ds(i * 16, 16)] = jnp.maximum(
                        xcon_v[pl.ds(i * 16, 16)] * wv + bv, 0.0)
                    return _
                lax.fori_loop(0, NCH, bdy, None)

            emit_row(1 + f, d, fill)

    for dj in range(2):
        d = wid + 32 * dj
        cv = plsc.load_gather(cls_v, [jnp.full((16,), d, dtype=jnp.int32)])

        def fill(buf, cv=cv):
            def bdy(i, _):
                buf[pl.ds(i * 16, 16)] = cv
                return _
            lax.fori_loop(0, NCH, bdy, None)

        emit_row(0, d, fill)

    idxbufs = [idx_a, idx_b]
    h_idx = [pltpu.async_copy(xcatT_hbm.at[0, :], idx_a, isem), None]
    for f in range(CAT):
        ib = idxbufs[f % 2]
        h_idx[f % 2].wait()
        if f + 1 < CAT:
            h_idx[(f + 1) % 2] = pltpu.async_copy(
                xcatT_hbm.at[f + 1, :], idxbufs[(f + 1) % 2], isem)
        for dj in range(2):
            d = wid + 32 * dj
            h_slab.wait()

            def fill(buf, ib=ib):
                def bdy(i, _):
                    buf[pl.ds(i * 16, 16)] = plsc.load_gather(
                        slab_v, [ib[pl.ds(i * 16, 16)]])
                    return _
                lax.fori_loop(0, NCH, bdy, None)

            emit_row(1 + CON + f, d, fill)
            nk = f * 2 + dj + 1
            if nk < 2 * CAT:
                nf, ndj = divmod(nk, 2)
                h_slab = pltpu.async_copy(
                    t_hbm.at[nf, wid + 32 * ndj, :], slab_v, ssem)

    for i in range(2):
        if pend[i] is not None:
            pend[i].wait()


@jax.jit
def _sc_call(xconF, xcatT, cls1, conW1, conb1, t3):
    mesh = plsc.VectorSubcoreMesh(core_axis_name="c", subcore_axis_name="s")
    kern = pl.kernel(
        _body,
        out_type=jax.ShapeDtypeStruct((ROWS, DIM, BATCH), jnp.float32),
        mesh=mesh,
        compiler_params=pltpu.CompilerParams(needs_layout_passes=False),
        scratch_types=[
            pltpu.VMEM((BATCH,), jnp.int32),
            pltpu.VMEM((BATCH,), jnp.int32),
            pltpu.VMEM((BATCH,), jnp.float32),
            pltpu.VMEM((DIM,), jnp.float32),
            pltpu.VMEM((CON * DIM,), jnp.float32),
            pltpu.VMEM((CON * DIM,), jnp.float32),
            pltpu.VMEM((VOCAB,), jnp.float32),
            pltpu.VMEM((BATCH,), jnp.float32),
            pltpu.VMEM((BATCH,), jnp.float32),
            pltpu.SemaphoreType.DMA,
            pltpu.SemaphoreType.DMA,
            pltpu.SemaphoreType.DMA,
        ],
    )
    return kern(xconF, xcatT, cls1, conW1, conb1, t3)


def kernel(x_con, x_cat, cls, con_W, con_b, tables):
    out = _sc_call(x_con.T.reshape(CON * BATCH), x_cat.astype(jnp.int32).T, cls.reshape(DIM),
                   con_W.reshape(CON * DIM), con_b.reshape(CON * DIM),
                   tables.transpose(0, 2, 1))
    return out.transpose(2, 0, 1)


# speedup vs baseline: 4.7512x; 1.0408x over previous
import jax
import jax.numpy as jnp
from jax import lax
from jax.experimental import pallas as pl
from jax.experimental.pallas import tpu as pltpu
from jax.experimental.pallas import tpu_sc as plsc

BATCH = 4096
CON = 13
CAT = 26
VOCAB = 100000
DIM = 64
ROWS = 1 + CON + CAT

NUM_CORES = 2
NUM_SUBCORES = 16
NW = NUM_CORES * NUM_SUBCORES
NCH = BATCH // 16
SPLIT = 50048
HI = VOCAB - SPLIT


def _body(xconF_hbm, xcatT_hbm, cls_hbm, conW_hbm, conb_hbm, t_hbm, out_hbm,
          idx_a, idx_b, xcon_v, cls_v, conW_v, conb_v, ha, hb, row_a, row_b,
          ssa, ssb, wsem, isem):
    wid = lax.axis_index("s") * NUM_CORES + lax.axis_index("c")

    pltpu.sync_copy(cls_hbm, cls_v)
    pltpu.sync_copy(conW_hbm, conW_v)
    pltpu.sync_copy(conb_hbm, conb_v)

    rowbufs = [row_a, row_b]
    pend = [None, None]
    state = {"t": 0}

    def emit_row(r, d, fill):
        i = state["t"]
        state["t"] ^= 1
        if pend[i] is not None:
            pend[i].wait()
        fill(rowbufs[i])
        pend[i] = pltpu.async_copy(rowbufs[i], out_hbm.at[r, d, :], wsem)

    h0 = pltpu.async_copy(t_hbm.at[0, wid, pl.ds(0, SPLIT)], ha, ssa)
    h1 = pltpu.async_copy(t_hbm.at[0, wid, pl.ds(SPLIT, HI)], hb, ssb)

    for f in range(CON):
        pltpu.sync_copy(xconF_hbm.at[pl.ds(f * BATCH, BATCH)], xcon_v)
        for dj in range(2):
            d = wid + 32 * dj
            wv = plsc.load_gather(
                conW_v, [jnp.full((16,), f * DIM + d, dtype=jnp.int32)])
            bv = plsc.load_gather(
                conb_v, [jnp.full((16,), f * DIM + d, dtype=jnp.int32)])

            def fill(buf, wv=wv, bv=bv):
                def bdy(i, _):
                    buf[pl.ds(i * 16, 16)] = jnp.maximum(
                        xcon_v[pl.ds(i * 16, 16)] * wv + bv, 0.0)
                    return _
                lax.fori_loop(0, NCH, bdy, None)

            emit_row(1 + f, d, fill)

    for dj in range(2):
        d = wid + 32 * dj
        cv = plsc.load_gather(cls_v, [jnp.full((16,), d, dtype=jnp.int32)])

        def fill(buf, cv=cv):
            def bdy(i, _):
                buf[pl.ds(i * 16, 16)] = cv
                return _
            lax.fori_loop(0, NCH, bdy, None)

        emit_row(0, d, fill)

    idxbufs = [idx_a, idx_b]
    h_idx = [pltpu.async_copy(xcatT_hbm.at[0, :], idx_a, isem), None]
    for f in range(CAT):
        ib = idxbufs[f % 2]
        h_idx[f % 2].wait()
        if f + 1 < CAT:
            h_idx[(f + 1) % 2] = pltpu.async_copy(
                xcatT_hbm.at[f + 1, :], idxbufs[(f + 1) % 2], isem)
        for dj in range(2):
            d = wid + 32 * dj
            nk = f * 2 + dj + 1
            nf, ndj = divmod(nk, 2)

            i = state["t"]
            state["t"] ^= 1
            if pend[i] is not None:
                pend[i].wait()
            buf = rowbufs[i]

            h0.wait()

            def pass_lo(ib=ib, buf=buf):
                def bdy(i, _):
                    iv = ib[pl.ds(i * 16, 16)]
                    m = iv < SPLIT
                    buf[pl.ds(i * 16, 16)] = plsc.load_gather(
                        ha, [iv], mask=m)
                    return _
                lax.fori_loop(0, NCH, bdy, None)

            pass_lo()
            if nk < 2 * CAT:
                h0 = pltpu.async_copy(
                    t_hbm.at[nf, wid + 32 * ndj, pl.ds(0, SPLIT)], ha, ssa)

            h1.wait()

            def pass_hi(ib=ib, buf=buf):
                def bdy(i, _):
                    iv = ib[pl.ds(i * 16, 16)]
                    m = iv >= SPLIT
                    g = plsc.load_gather(hb, [iv - SPLIT], mask=m)
                    buf[pl.ds(i * 16, 16)] = jnp.where(
                        m, g, buf[pl.ds(i * 16, 16)])
                    return _
                lax.fori_loop(0, NCH, bdy, None)

            pass_hi()
            if nk < 2 * CAT:
                h1 = pltpu.async_copy(
                    t_hbm.at[nf, wid + 32 * ndj, pl.ds(SPLIT, HI)], hb, ssb)

            pend[i] = pltpu.async_copy(buf, out_hbm.at[1 + CON + f, d, :],
                                       wsem)

    for i in range(2):
        if pend[i] is not None:
            pend[i].wait()


@jax.jit
def _sc_call(xconF, xcatT, cls1, conW1, conb1, t3):
    mesh = plsc.VectorSubcoreMesh(core_axis_name="c", subcore_axis_name="s")
    kern = pl.kernel(
        _body,
        out_type=jax.ShapeDtypeStruct((ROWS, DIM, BATCH), jnp.float32),
        mesh=mesh,
        compiler_params=pltpu.CompilerParams(needs_layout_passes=False),
        scratch_types=[
            pltpu.VMEM((BATCH,), jnp.int32),
            pltpu.VMEM((BATCH,), jnp.int32),
            pltpu.VMEM((BATCH,), jnp.float32),
            pltpu.VMEM((DIM,), jnp.float32),
            pltpu.VMEM((CON * DIM,), jnp.float32),
            pltpu.VMEM((CON * DIM,), jnp.float32),
            pltpu.VMEM((SPLIT,), jnp.float32),
            pltpu.VMEM((HI,), jnp.float32),
            pltpu.VMEM((BATCH,), jnp.float32),
            pltpu.VMEM((BATCH,), jnp.float32),
            pltpu.SemaphoreType.DMA,
            pltpu.SemaphoreType.DMA,
            pltpu.SemaphoreType.DMA,
            pltpu.SemaphoreType.DMA,
        ],
    )
    return kern(xconF, xcatT, cls1, conW1, conb1, t3)


def kernel(x_con, x_cat, cls, con_W, con_b, tables):
    out = _sc_call(x_con.T.reshape(CON * BATCH), x_cat.astype(jnp.int32).T, cls.reshape(DIM),
                   con_W.reshape(CON * DIM), con_b.reshape(CON * DIM),
                   tables.transpose(0, 2, 1))
    return out.transpose(2, 0, 1)


# speedup vs baseline: 5.0073x; 1.0539x over previous
import jax
import jax.numpy as jnp
from jax import lax
from jax.experimental import pallas as pl
from jax.experimental.pallas import tpu as pltpu
from jax.experimental.pallas import tpu_sc as plsc

BATCH = 4096
CON = 13
CAT = 26
VOCAB = 100000
DIM = 64
ROWS = 1 + CON + CAT

NUM_CORES = 2
NUM_SUBCORES = 16
NW = NUM_CORES * NUM_SUBCORES
NCH = BATCH // 16
SPLIT = 50048
HI = VOCAB - SPLIT


def _body(xconF_hbm, xcatT_hbm, cls_hbm, conW_hbm, conb_hbm, t_hbm, out_hbm,
          idx_a, idx_b, xcon_v, cls_v, conW_v, conb_v, ha, hb, row_a, row_b,
          ssa, ssb, wsem, isem):
    wid = lax.axis_index("s") * NUM_CORES + lax.axis_index("c")

    pltpu.sync_copy(cls_hbm, cls_v)
    pltpu.sync_copy(conW_hbm, conW_v)
    pltpu.sync_copy(conb_hbm, conb_v)

    rowbufs = [row_a, row_b]
    pend = [None, None]
    state = {"t": 0}

    def emit_row(r, d, fill):
        i = state["t"]
        state["t"] ^= 1
        if pend[i] is not None:
            pend[i].wait()
        fill(rowbufs[i])
        pend[i] = pltpu.async_copy(rowbufs[i], out_hbm.at[r, d, :], wsem)

    h0 = pltpu.async_copy(t_hbm.at[0, wid, pl.ds(0, SPLIT)], ha, ssa)
    h1 = pltpu.async_copy(t_hbm.at[0, wid, pl.ds(SPLIT, HI)], hb, ssb)

    dense_state = {"f": -1}

    def emit_dense(task):
        if task >= 2 * CON:
            dj = task - 2 * CON
            d = wid + 32 * dj
            cv = plsc.load_gather(cls_v, [jnp.full((16,), d, jnp.int32)])

            def fill(buf, cv=cv):
                def bdy(i, _):
                    buf[pl.ds(i * 16, 16)] = cv
                    return _
                lax.fori_loop(0, NCH, bdy, None)

            emit_row(0, d, fill)
            return
        f, dj = divmod(task, 2)
        if dense_state["f"] != f:
            pltpu.sync_copy(xconF_hbm.at[pl.ds(f * BATCH, BATCH)], xcon_v)
            dense_state["f"] = f
        d = wid + 32 * dj
        wv = plsc.load_gather(
            conW_v, [jnp.full((16,), f * DIM + d, dtype=jnp.int32)])
        bv = plsc.load_gather(
            conb_v, [jnp.full((16,), f * DIM + d, dtype=jnp.int32)])

        def fill(buf, wv=wv, bv=bv):
            def bdy(i, _):
                buf[pl.ds(i * 16, 16)] = jnp.maximum(
                    xcon_v[pl.ds(i * 16, 16)] * wv + bv, 0.0)
                return _
            lax.fori_loop(0, NCH, bdy, None)

        emit_row(1 + f, d, fill)

    NDENSE = 2 * CON + 2
    NPRE = 10
    for task in range(NPRE):
        emit_dense(task)

    idxbufs = [idx_a, idx_b]
    h_idx = [pltpu.async_copy(xcatT_hbm.at[0, :], idx_a, isem), None]
    for f in range(CAT):
        ib = idxbufs[f % 2]
        h_idx[f % 2].wait()
        if f + 1 < CAT:
            h_idx[(f + 1) % 2] = pltpu.async_copy(
                xcatT_hbm.at[f + 1, :], idxbufs[(f + 1) % 2], isem)
        for dj in range(2):
            d = wid + 32 * dj
            nk = f * 2 + dj + 1
            nf, ndj = divmod(nk, 2)

            i = state["t"]
            state["t"] ^= 1
            if pend[i] is not None:
                pend[i].wait()
            buf = rowbufs[i]

            h0.wait()

            def pass_lo(ib=ib, buf=buf):
                def bdy(i, _):
                    iv = ib[pl.ds(i * 16, 16)]
                    m = iv < SPLIT
                    buf[pl.ds(i * 16, 16)] = plsc.load_gather(
                        ha, [iv], mask=m)
                    return _
                lax.fori_loop(0, NCH, bdy, None)

            pass_lo()
            if nk < 2 * CAT:
                h0 = pltpu.async_copy(
                    t_hbm.at[nf, wid + 32 * ndj, pl.ds(0, SPLIT)], ha, ssa)

            h1.wait()

            def pass_hi(ib=ib, buf=buf):
                def bdy(i, _):
                    iv = ib[pl.ds(i * 16, 16)]
                    m = iv >= SPLIT
                    g = plsc.load_gather(hb, [iv - SPLIT], mask=m)
                    buf[pl.ds(i * 16, 16)] = jnp.where(
                        m, g, buf[pl.ds(i * 16, 16)])
                    return _
                lax.fori_loop(0, NCH, bdy, None)

            pass_hi()
            if nk < 2 * CAT:
                h1 = pltpu.async_copy(
                    t_hbm.at[nf, wid + 32 * ndj, pl.ds(SPLIT, HI)], hb, ssb)

            pend[i] = pltpu.async_copy(buf, out_hbm.at[1 + CON + f, d, :],
                                       wsem)
            step = f * 2 + dj
            if NPRE + step < NDENSE:
                emit_dense(NPRE + step)

    for i in range(2):
        if pend[i] is not None:
            pend[i].wait()


@jax.jit
def _sc_call(xconF, xcatT, cls1, conW1, conb1, t3):
    mesh = plsc.VectorSubcoreMesh(core_axis_name="c", subcore_axis_name="s")
    kern = pl.kernel(
        _body,
        out_type=jax.ShapeDtypeStruct((ROWS, DIM, BATCH), jnp.float32),
        mesh=mesh,
        compiler_params=pltpu.CompilerParams(needs_layout_passes=False),
        scratch_types=[
            pltpu.VMEM((BATCH,), jnp.int32),
            pltpu.VMEM((BATCH,), jnp.int32),
            pltpu.VMEM((BATCH,), jnp.float32),
            pltpu.VMEM((DIM,), jnp.float32),
            pltpu.VMEM((CON * DIM,), jnp.float32),
            pltpu.VMEM((CON * DIM,), jnp.float32),
            pltpu.VMEM((SPLIT,), jnp.float32),
            pltpu.VMEM((HI,), jnp.float32),
            pltpu.VMEM((BATCH,), jnp.float32),
            pltpu.VMEM((BATCH,), jnp.float32),
            pltpu.SemaphoreType.DMA,
            pltpu.SemaphoreType.DMA,
            pltpu.SemaphoreType.DMA,
            pltpu.SemaphoreType.DMA,
        ],
    )
    return kern(xconF, xcatT, cls1, conW1, conb1, t3)


def kernel(x_con, x_cat, cls, con_W, con_b, tables):
    out = _sc_call(x_con.T.reshape(CON * BATCH), x_cat.astype(jnp.int32).T, cls.reshape(DIM),
                   con_W.reshape(CON * DIM), con_b.reshape(CON * DIM),
                   tables.transpose(0, 2, 1))
    return out.transpose(2, 0, 1)
